# baseline (speedup 1.0000x reference)
import functools

import jax
import jax.numpy as jnp
from jax import lax
from jax.experimental import pallas as pl
from jax.experimental.pallas import tpu as pltpu
from jax.experimental.pallas import tpu_sc as plsc

B, S, D = 4, 2048, 768
E, K = 16, 2
DH = int(D * 4 * 2 / 3)
N = B * S
CAP = int(N * K / E * 1.25)
THRESH = 0.2

TBLK = 512
NTB = N // TBLK
FTILE = 256
NROWS = (E + 1) * CAP
TRASH = E * CAP
D2 = D // 2

NW = 32
TPW = N // NW
DCH = 128
CCH = 32


def _router_body(x_ref, wg_ref, meta_ref, stats_ref, w0s_ref, w1s_ref,
                 xn_ref, cnt_ref):
    t = pl.program_id(0)

    @pl.when(t == 0)
    def _init():
        cnt_ref[...] = jnp.zeros_like(cnt_ref)
        stats_ref[...] = jnp.zeros_like(stats_ref)

    xb = x_ref[...]
    logits = jnp.dot(xb, wg_ref[...], preferred_element_type=jnp.float32)
    m = jnp.max(logits, axis=1, keepdims=True)
    ex = jnp.exp(logits - m)
    sumex = jnp.sum(ex, axis=1, keepdims=True)
    lse = jnp.log(sumex) + m
    probs = ex / sumex

    mu = jnp.mean(xb, axis=1, keepdims=True)
    xc = xb - mu
    var = jnp.mean(xc * xc, axis=1, keepdims=True)
    xn_ref[...] = (xc * lax.rsqrt(var + 1e-5)).astype(jnp.bfloat16)

    lane = lax.broadcasted_iota(jnp.int32, (TBLK, E), 1)
    g0 = jnp.max(probs, axis=1, keepdims=True)
    i0 = jnp.min(jnp.where(probs == g0, lane, E), axis=1, keepdims=True)
    c0 = lane == i0
    probs2 = jnp.where(c0, -jnp.inf, probs)
    g1 = jnp.max(probs2, axis=1, keepdims=True)
    i1 = jnp.min(jnp.where(probs2 == g1, lane, E), axis=1, keepdims=True)
    c1 = lane == i1
    g1k = jnp.where(g1 > THRESH, g1, 0.0)

    c0f = c0.astype(jnp.float32)
    c1f = c1.astype(jnp.float32)
    csum = c0f + c1f
    row = lax.broadcasted_iota(jnp.int32, (TBLK, TBLK), 0)
    col = lax.broadcasted_iota(jnp.int32, (TBLK, TBLK), 1)
    tril = (col < row).astype(jnp.float32)
    scum = jnp.dot(tril, csum, preferred_element_type=jnp.float32)
    base = scum + cnt_ref[0:1, :]
    pos0 = jnp.sum(jnp.where(c0, base, 0.0), axis=1, keepdims=True)
    pos1 = jnp.sum(jnp.where(c1, base, 0.0), axis=1, keepdims=True)

    keep0 = pos0 < CAP
    keep1 = pos1 < CAP
    i0f = i0.astype(jnp.float32)
    i1f = i1.astype(jnp.float32)
    w0 = g0 * keep0.astype(jnp.float32)
    w1 = g1k * keep1.astype(jnp.float32)
    dst0 = jnp.where(keep0, i0f * CAP + pos0, float(TRASH))
    dst1 = jnp.where(keep1, i1f * CAP + pos1, float(TRASH))

    zero = jnp.zeros_like(pos0)
    meta_ref[...] = jnp.concatenate(
        [dst0, dst1, zero, zero, zero, zero, zero, zero], axis=1)
    w0s_ref[...] = jnp.broadcast_to(w0, (TBLK, 128))
    w1s_ref[...] = jnp.broadcast_to(w1, (TBLK, 128))

    cnt_ref[0:1, :] += jnp.sum(csum, axis=0, keepdims=True)
    stats_ref[0:1, :] += jnp.sum(probs, axis=0, keepdims=True)
    stats_ref[1:2, :] += jnp.sum(c0f, axis=0, keepdims=True)
    zpart = jnp.sum(lse * lse)
    stats_ref[2:3, :] += jnp.broadcast_to(zpart, (1, E))


_router = pl.pallas_call(
    _router_body,
    grid=(NTB,),
    in_specs=[
        pl.BlockSpec((TBLK, D), lambda i: (i, 0)),
        pl.BlockSpec((D, E), lambda i: (0, 0)),
    ],
    out_specs=[
        pl.BlockSpec((TBLK, 8), lambda i: (i, 0)),
        pl.BlockSpec((8, E), lambda i: (0, 0)),
        pl.BlockSpec((TBLK, 128), lambda i: (i, 0)),
        pl.BlockSpec((TBLK, 128), lambda i: (i, 0)),
        pl.BlockSpec((TBLK, D), lambda i: (i, 0)),
    ],
    out_shape=[
        jax.ShapeDtypeStruct((N, 8), jnp.float32),
        jax.ShapeDtypeStruct((8, E), jnp.float32),
        jax.ShapeDtypeStruct((N, 128), jnp.float32),
        jax.ShapeDtypeStruct((N, 128), jnp.float32),
        jax.ShapeDtypeStruct((N, D), jnp.bfloat16),
    ],
    scratch_shapes=[pltpu.VMEM((8, E), jnp.float32)],
)


def _dispatch_body(x_hbm, d0_hbm, d1_hbm, w0_hbm, w1_hbm, buf_hbm, gbuf_hbm,
                   d0v, d1v, rowsA, rowsB, gv, sem, gsem):
    wid = lax.axis_index("s") * 2 + lax.axis_index("c")
    nch = TPW // DCH
    pltpu.sync_copy(d0_hbm.at[pl.ds(wid * nch, nch)], d0v)
    pltpu.sync_copy(d1_hbm.at[pl.ds(wid * nch, nch)], d1v)
    rows = (rowsA, rowsB)
    pltpu.sync_copy(x_hbm.at[pl.ds(wid * TPW, DCH)], rowsA)
    for j in range(nch):
        r = rows[j % 2]
        cps = [
            pltpu.async_copy(r, buf_hbm.at[d0v.at[j]], sem),
            pltpu.async_copy(r, buf_hbm.at[d1v.at[j]], sem),
        ]
        pltpu.sync_copy(w0_hbm.at[pl.ds(wid * TPW + j * DCH, DCH)], gv)
        pltpu.async_copy(gv, gbuf_hbm.at[d0v.at[j]], gsem).wait()
        pltpu.sync_copy(w1_hbm.at[pl.ds(wid * TPW + j * DCH, DCH)], gv)
        cps.append(pltpu.async_copy(gv, gbuf_hbm.at[d1v.at[j]], gsem))
        if j + 1 < nch:
            pltpu.sync_copy(
                x_hbm.at[pl.ds(wid * TPW + (j + 1) * DCH, DCH)], rows[(j + 1) % 2])
        for cp in cps:
            cp.wait()


def _ffn_body(lng_ref, lnb_ref, b1_ref, b2_ref, x_ref, gs_ref,
              w1_ref, w2_ref, o_ref):
    e = pl.program_id(0)

    @pl.when(e < E)
    def _compute():
        for j in range(CAP // FTILE):
            sl = pl.ds(j * FTILE, FTILE)
            xn = x_ref[sl, :].astype(jnp.float32)
            hh = xn * lng_ref[0] + lnb_ref[0]
            h1 = jnp.dot(hh.astype(jnp.bfloat16),
                         w1_ref[0].astype(jnp.bfloat16),
                         preferred_element_type=jnp.float32) + b1_ref[0]
            h1 = jnp.where(h1 >= 0, h1, 0.01 * h1)
            y = jnp.dot(h1.astype(jnp.bfloat16),
                        w2_ref[0].astype(jnp.bfloat16),
                        preferred_element_type=jnp.float32) + b2_ref[0]
            o_ref[sl, :] = y * gs_ref[sl, 0:1]

    @pl.when(e == E)
    def _zero():
        o_ref[...] = jnp.zeros_like(o_ref)


_ffn = pl.pallas_call(
    _ffn_body,
    grid=(E + 1,),
    in_specs=[
        pl.BlockSpec((1, 1, D), lambda e: (jnp.minimum(e, E - 1), 0, 0)),
        pl.BlockSpec((1, 1, D), lambda e: (jnp.minimum(e, E - 1), 0, 0)),
        pl.BlockSpec((1, 1, DH), lambda e: (jnp.minimum(e, E - 1), 0, 0)),
        pl.BlockSpec((1, 1, D), lambda e: (jnp.minimum(e, E - 1), 0, 0)),
        pl.BlockSpec((CAP, D), lambda e: (e, 0)),
        pl.BlockSpec((CAP, 128), lambda e: (e, 0)),
        pl.BlockSpec((1, D, DH), lambda e: (jnp.minimum(e, E - 1), 0, 0)),
        pl.BlockSpec((1, DH, D), lambda e: (jnp.minimum(e, E - 1), 0, 0)),
    ],
    out_specs=pl.BlockSpec((CAP, D), lambda e: (e, 0)),
    out_shape=jax.ShapeDtypeStruct((NROWS, D), jnp.float32),
)


def _combine_body(h_hbm, s0_hbm, s1_hbm, out_hbm,
                  s0v, s1v, r0A, r1A, r0B, r1B, semA, semB):
    wid = lax.axis_index("s") * 2 + lax.axis_index("c")
    base = wid * TPW
    nch = TPW // CCH
    pltpu.sync_copy(s0_hbm.at[pl.ds(base, TPW)], s0v)
    pltpu.sync_copy(s1_hbm.at[pl.ds(base, TPW)], s1v)
    bufs = ((r0A, r1A, semA), (r0B, r1B, semB))

    def start(c):
        r0, r1, sem = bufs[c % 2]
        return (pltpu.async_copy(h_hbm.at[s0v.at[pl.ds(c * CCH, CCH)]], r0, sem),
                pltpu.async_copy(h_hbm.at[s1v.at[pl.ds(c * CCH, CCH)]], r1, sem))

    pend = start(0)
    for c in range(nch):
        r0, r1, _ = bufs[c % 2]
        for cp in pend:
            cp.wait()
        if c + 1 < nch:
            pend = start(c + 1)

        def tok_body(t, carry):
            for dd in range(D // 16):
                sl = pl.ds(dd * 16, 16)
                r0[t, sl] = r0[t, sl] + r1[t, sl]
            return carry

        lax.fori_loop(0, CCH, tok_body, 0)
        pltpu.sync_copy(r0, out_hbm.at[pl.ds(base + c * CCH, CCH)])


@functools.lru_cache(maxsize=1)
def _sc_kernels():
    mesh = plsc.VectorSubcoreMesh(
        core_axis_name="c", subcore_axis_name="s", num_cores=2, num_subcores=16)
    dispatch = pl.kernel(
        _dispatch_body,
        out_type=[
            jax.ShapeDtypeStruct((NROWS, D2), jnp.int32),
            jax.ShapeDtypeStruct((NROWS, 128), jnp.float32),
        ],
        mesh=mesh,
        scratch_types=[
            pltpu.VMEM((TPW // DCH, DCH), jnp.int32),
            pltpu.VMEM((TPW // DCH, DCH), jnp.int32),
            pltpu.VMEM((DCH, D2), jnp.int32),
            pltpu.VMEM((DCH, D2), jnp.int32),
            pltpu.VMEM((DCH, 128), jnp.float32),
            pltpu.SemaphoreType.DMA,
            pltpu.SemaphoreType.DMA,
        ],
    )
    combine = pl.kernel(
        _combine_body,
        out_type=jax.ShapeDtypeStruct((N, D), jnp.float32),
        mesh=mesh,
        scratch_types=[
            pltpu.VMEM((TPW,), jnp.int32),
            pltpu.VMEM((TPW,), jnp.int32),
            pltpu.VMEM((CCH, D), jnp.float32),
            pltpu.VMEM((CCH, D), jnp.float32),
            pltpu.VMEM((CCH, D), jnp.float32),
            pltpu.VMEM((CCH, D), jnp.float32),
            pltpu.SemaphoreType.DMA,
            pltpu.SemaphoreType.DMA,
        ],
    )
    return dispatch, combine


def kernel(x, Wg, ln_g, ln_b, W1, b1, W2, b2):
    _dispatch, _combine = _sc_kernels()
    xf = x.reshape(N, D)
    meta, stats, w0s, w1s, xn = _router(xf, Wg)
    dst0 = meta[:, 0].astype(jnp.int32)
    dst1 = meta[:, 1].astype(jnp.int32)
    d0 = dst0.reshape(N // DCH, DCH)
    d1 = dst1.reshape(N // DCH, DCH)

    me = stats[0, :] / N
    ce = stats[1, :] / N
    z_loss = stats[2, 0] / N
    balance = E * jnp.sum(me * ce)
    total = 0.01 * balance + 0.001 * z_loss

    xn_i32 = lax.bitcast_convert_type(xn.reshape(N, D2, 2), jnp.int32)
    buf_i32, gbuf = _dispatch(xn_i32, d0, d1, w0s, w1s)
    buf = lax.bitcast_convert_type(buf_i32, jnp.bfloat16).reshape(NROWS, D)
    h = _ffn(ln_g.reshape(E, 1, D), ln_b.reshape(E, 1, D),
             b1.reshape(E, 1, DH), b2.reshape(E, 1, D), buf, gbuf, W1, W2)
    out = _combine(h, dst0, dst1)
    return out.reshape(B, S, D), total, balance, z_loss


# speedup vs baseline: 2.0609x; 2.0609x over previous
import functools

import jax
import jax.numpy as jnp
from jax import lax
from jax.experimental import pallas as pl
from jax.experimental.pallas import tpu as pltpu
from jax.experimental.pallas import tpu_sc as plsc

B, S, D = 4, 2048, 768
E, K = 16, 2
DH = int(D * 4 * 2 / 3)
N = B * S
CAP = int(N * K / E * 1.25)
THRESH = 0.2

TBLK = 1024
NTB = N // TBLK
FTILE = 256
NROWS = E * CAP + FTILE
NBLK = NROWS // FTILE
TRASH = E * CAP

NW = 32
TPW = N // NW
DCH = 128
CCH = 32


def _router_body(x_ref, wg_ref, d0_ref, d1_ref, stats_ref, w0s_ref, w1s_ref,
                 xn_ref, cnt_ref):
    t = pl.program_id(0)

    @pl.when(t == 0)
    def _init():
        cnt_ref[...] = jnp.zeros_like(cnt_ref)
        stats_ref[...] = jnp.zeros_like(stats_ref)

    xb = x_ref[...]
    logits = jnp.dot(xb, wg_ref[...], preferred_element_type=jnp.float32)
    m = jnp.max(logits, axis=1, keepdims=True)
    ex = jnp.exp(logits - m)
    sumex = jnp.sum(ex, axis=1, keepdims=True)
    lse = jnp.log(sumex) + m
    probs = ex / sumex

    mu = jnp.mean(xb, axis=1, keepdims=True)
    xc = xb - mu
    var = jnp.mean(xc * xc, axis=1, keepdims=True)
    xn_ref[...] = xc * lax.rsqrt(var + 1e-5)

    lane = lax.broadcasted_iota(jnp.int32, (TBLK, E), 1)
    g0 = jnp.max(probs, axis=1, keepdims=True)
    i0 = jnp.min(jnp.where(probs == g0, lane, E), axis=1, keepdims=True)
    c0 = lane == i0
    probs2 = jnp.where(c0, -jnp.inf, probs)
    g1 = jnp.max(probs2, axis=1, keepdims=True)
    i1 = jnp.min(jnp.where(probs2 == g1, lane, E), axis=1, keepdims=True)
    c1 = lane == i1
    g1k = jnp.where(g1 > THRESH, g1, 0.0)

    c0f = c0.astype(jnp.float32)
    c1f = c1.astype(jnp.float32)
    csum = c0f + c1f
    row = lax.broadcasted_iota(jnp.int32, (TBLK, TBLK), 0)
    col = lax.broadcasted_iota(jnp.int32, (TBLK, TBLK), 1)
    tril = (col < row).astype(jnp.float32)
    scum = jnp.dot(tril, csum, preferred_element_type=jnp.float32)
    base = scum + cnt_ref[0:1, :]
    pos0 = jnp.sum(jnp.where(c0, base, 0.0), axis=1, keepdims=True)
    pos1 = jnp.sum(jnp.where(c1, base, 0.0), axis=1, keepdims=True)

    keep0 = pos0 < CAP
    keep1 = pos1 < CAP
    i0f = i0.astype(jnp.float32)
    i1f = i1.astype(jnp.float32)
    w0 = g0 * keep0.astype(jnp.float32)
    w1 = g1k * keep1.astype(jnp.float32)
    dst0 = jnp.where(keep0, i0f * CAP + pos0, float(TRASH))
    dst1 = jnp.where(keep1, i1f * CAP + pos1, float(TRASH))

    d0_ref[...] = dst0.astype(jnp.int32).reshape(TBLK // 128, 128)
    d1_ref[...] = dst1.astype(jnp.int32).reshape(TBLK // 128, 128)
    w0s_ref[...] = jnp.broadcast_to(w0, (TBLK, 128))
    w1s_ref[...] = jnp.broadcast_to(w1, (TBLK, 128))

    cnt_ref[0:1, :] += jnp.sum(csum, axis=0, keepdims=True)
    stats_ref[0:1, :] += jnp.sum(probs, axis=0, keepdims=True)
    stats_ref[1:2, :] += jnp.sum(c0f, axis=0, keepdims=True)
    zpart = jnp.sum(lse * lse)
    stats_ref[2:3, :] += jnp.broadcast_to(zpart, (1, E))


_router = pl.pallas_call(
    _router_body,
    grid=(NTB,),
    in_specs=[
        pl.BlockSpec((TBLK, D), lambda i: (i, 0)),
        pl.BlockSpec((D, E), lambda i: (0, 0)),
    ],
    out_specs=[
        pl.BlockSpec((TBLK // 128, 128), lambda i: (i, 0)),
        pl.BlockSpec((TBLK // 128, 128), lambda i: (i, 0)),
        pl.BlockSpec((8, E), lambda i: (0, 0)),
        pl.BlockSpec((TBLK, 128), lambda i: (i, 0)),
        pl.BlockSpec((TBLK, 128), lambda i: (i, 0)),
        pl.BlockSpec((TBLK, D), lambda i: (i, 0)),
    ],
    out_shape=[
        jax.ShapeDtypeStruct((N // 128, 128), jnp.int32),
        jax.ShapeDtypeStruct((N // 128, 128), jnp.int32),
        jax.ShapeDtypeStruct((8, E), jnp.float32),
        jax.ShapeDtypeStruct((N, 128), jnp.float32),
        jax.ShapeDtypeStruct((N, 128), jnp.float32),
        jax.ShapeDtypeStruct((N, D), jnp.float32),
    ],
    scratch_shapes=[pltpu.VMEM((8, E), jnp.float32)],
)


def _dispatch_body(x_hbm, d0_hbm, d1_hbm, w0_hbm, w1_hbm, buf_hbm, gbuf_hbm,
                   d0v, d1v, rows, gv, sem, gsem):
    wid = lax.axis_index("s") * 2 + lax.axis_index("c")
    nch = TPW // DCH
    pltpu.sync_copy(d0_hbm.at[pl.ds(wid * nch, nch)], d0v)
    pltpu.sync_copy(d1_hbm.at[pl.ds(wid * nch, nch)], d1v)
    for j in range(nch):
        pltpu.sync_copy(x_hbm.at[pl.ds(wid * TPW + j * DCH, DCH)], rows)
        cps = [
            pltpu.async_copy(rows, buf_hbm.at[d0v.at[j]], sem),
            pltpu.async_copy(rows, buf_hbm.at[d1v.at[j]], sem),
        ]
        pltpu.sync_copy(w0_hbm.at[pl.ds(wid * TPW + j * DCH, DCH)], gv)
        pltpu.async_copy(gv, gbuf_hbm.at[d0v.at[j]], gsem).wait()
        pltpu.sync_copy(w1_hbm.at[pl.ds(wid * TPW + j * DCH, DCH)], gv)
        cps.append(pltpu.async_copy(gv, gbuf_hbm.at[d1v.at[j]], gsem))
        for cp in cps:
            cp.wait()


def _ffn_body(lng_ref, lnb_ref, b1_ref, b2_ref, x_ref, gs_ref,
              w1_ref, w2_ref, o_ref):
    i = pl.program_id(0)

    @pl.when(i < NBLK - 1)
    def _compute():
        xn = x_ref[...]
        hh = xn * lng_ref[0] + lnb_ref[0]
        h1 = jnp.dot(hh.astype(jnp.bfloat16), w1_ref[0].astype(jnp.bfloat16),
                     preferred_element_type=jnp.float32) + b1_ref[0]
        h1 = jnp.where(h1 >= 0, h1, 0.01 * h1)
        y = jnp.dot(h1.astype(jnp.bfloat16), w2_ref[0].astype(jnp.bfloat16),
                    preferred_element_type=jnp.float32) + b2_ref[0]
        o_ref[...] = y * gs_ref[:, 0:1]

    @pl.when(i == NBLK - 1)
    def _zero():
        o_ref[...] = jnp.zeros_like(o_ref)


_ffn = pl.pallas_call(
    _ffn_body,
    grid=(NBLK,),
    in_specs=[
        pl.BlockSpec((1, 1, D), lambda i: (jnp.minimum(i // 5, E - 1), 0, 0)),
        pl.BlockSpec((1, 1, D), lambda i: (jnp.minimum(i // 5, E - 1), 0, 0)),
        pl.BlockSpec((1, 1, DH), lambda i: (jnp.minimum(i // 5, E - 1), 0, 0)),
        pl.BlockSpec((1, 1, D), lambda i: (jnp.minimum(i // 5, E - 1), 0, 0)),
        pl.BlockSpec((FTILE, D), lambda i: (i, 0)),
        pl.BlockSpec((FTILE, 128), lambda i: (i, 0)),
        pl.BlockSpec((1, D, DH), lambda i: (jnp.minimum(i // 5, E - 1), 0, 0)),
        pl.BlockSpec((1, DH, D), lambda i: (jnp.minimum(i // 5, E - 1), 0, 0)),
    ],
    out_specs=pl.BlockSpec((FTILE, D), lambda i: (i, 0)),
    out_shape=jax.ShapeDtypeStruct((NROWS, D), jnp.float32),
)


def _combine_body(h_hbm, s0_hbm, s1_hbm, out_hbm,
                  s0v, s1v, r0A, r1A, r0B, r1B, semA, semB):
    wid = lax.axis_index("s") * 2 + lax.axis_index("c")
    base = wid * TPW
    nch = TPW // CCH
    nrow = TPW // 128
    pltpu.sync_copy(s0_hbm.at[pl.ds(wid * nrow, nrow)], s0v)
    pltpu.sync_copy(s1_hbm.at[pl.ds(wid * nrow, nrow)], s1v)
    bufs = ((r0A, r1A, semA), (r0B, r1B, semB))
    per_row = 128 // CCH

    def start(c):
        r0, r1, sem = bufs[c % 2]
        rr = c // per_row
        off = (c % per_row) * CCH
        return (pltpu.async_copy(h_hbm.at[s0v.at[rr, pl.ds(off, CCH)]], r0, sem),
                pltpu.async_copy(h_hbm.at[s1v.at[rr, pl.ds(off, CCH)]], r1, sem))

    pend = start(0)
    for c in range(nch):
        r0, r1, _ = bufs[c % 2]
        for cp in pend:
            cp.wait()
        if c + 1 < nch:
            pend = start(c + 1)

        def tok_body(t, carry):
            for dd in range(D // 16):
                sl = pl.ds(dd * 16, 16)
                r0[t, sl] = r0[t, sl] + r1[t, sl]
            return carry

        lax.fori_loop(0, CCH, tok_body, 0)
        pltpu.sync_copy(r0, out_hbm.at[pl.ds(base + c * CCH, CCH)])


@functools.lru_cache(maxsize=1)
def _sc_kernels():
    mesh = plsc.VectorSubcoreMesh(
        core_axis_name="c", subcore_axis_name="s", num_cores=2, num_subcores=16)
    dispatch = pl.kernel(
        _dispatch_body,
        out_type=[
            jax.ShapeDtypeStruct((NROWS, D), jnp.float32),
            jax.ShapeDtypeStruct((NROWS, 128), jnp.float32),
        ],
        mesh=mesh,
        scratch_types=[
            pltpu.VMEM((TPW // DCH, DCH), jnp.int32),
            pltpu.VMEM((TPW // DCH, DCH), jnp.int32),
            pltpu.VMEM((DCH, D), jnp.float32),
            pltpu.VMEM((DCH, 128), jnp.float32),
            pltpu.SemaphoreType.DMA,
            pltpu.SemaphoreType.DMA,
        ],
    )
    combine = pl.kernel(
        _combine_body,
        out_type=jax.ShapeDtypeStruct((N, D), jnp.float32),
        mesh=mesh,
        scratch_types=[
            pltpu.VMEM((TPW // 128, 128), jnp.int32),
            pltpu.VMEM((TPW // 128, 128), jnp.int32),
            pltpu.VMEM((CCH, D), jnp.float32),
            pltpu.VMEM((CCH, D), jnp.float32),
            pltpu.VMEM((CCH, D), jnp.float32),
            pltpu.VMEM((CCH, D), jnp.float32),
            pltpu.SemaphoreType.DMA,
            pltpu.SemaphoreType.DMA,
        ],
    )
    return dispatch, combine


def kernel(x, Wg, ln_g, ln_b, W1, b1, W2, b2):
    _dispatch, _combine = _sc_kernels()
    xf = x.reshape(N, D)
    d0, d1, stats, w0s, w1s, xn = _router(xf, Wg)

    me = stats[0, :] / N
    ce = stats[1, :] / N
    z_loss = stats[2, 0] / N
    balance = E * jnp.sum(me * ce)
    total = 0.01 * balance + 0.001 * z_loss

    buf, gbuf = _dispatch(xn, d0, d1, w0s, w1s)
    h = _ffn(ln_g.reshape(E, 1, D), ln_b.reshape(E, 1, D),
             b1.reshape(E, 1, DH), b2.reshape(E, 1, D), buf, gbuf, W1, W2)
    out = _combine(h, d0, d1)
    return out.reshape(B, S, D), total, balance, z_loss
